# baseline (speedup 1.0000x reference)
import functools

import jax
import jax.numpy as jnp
from jax import lax
from jax.experimental import pallas as pl
from jax.experimental.pallas import tpu as pltpu
from jax.experimental.pallas import tpu_sc as plsc

N, E, D, DFF = 10000, 320000, 128, 512
NC, NS = 2, 16
NW = NC * NS
NSL = 2
E2 = E // NSL
EPWG = E // NW
CH = 80
NCH = EPWG // CH
GRP = 5
NGRP = NCH // GRP
EPW = E2 // NW
CHS = 40
NCHS = EPW // CHS
NPAIR = (NCHS - 1) // 2
EPWD = E // NW
CHD = 80
NCHD = EPWD // CHD
DW = 128
RPW = 1000
NRW = N // RPW

_SQRT_HALF = 0.7071067811865476


def _gelu(v):
    return 0.5 * v * (1.0 + lax.erf(v * _SQRT_HALF))


def _xa_body(x_ref, w_ref, o_ref):
    o_ref[...] = jnp.dot(x_ref[...], w_ref[...],
                         preferred_element_type=jnp.float32)


def _tc_xa(x, w1a):
    bn = 2000
    return pl.pallas_call(
        _xa_body,
        grid=(N // bn,),
        in_specs=[pl.BlockSpec((bn, D), lambda i: (i, 0)),
                  pl.BlockSpec((D, D), lambda i: (0, 0))],
        out_specs=pl.BlockSpec((bn, D), lambda i: (i, 0)),
        out_shape=jax.ShapeDtypeStruct((N, D), jnp.float32),
    )(x, w1a)


def _sc_gather(xa, idx3):
    mesh = plsc.VectorSubcoreMesh(core_axis_name="c", subcore_axis_name="s",
                                  num_cores=NC, num_subcores=NS)

    @functools.partial(
        pl.kernel,
        out_type=jax.ShapeDtypeStruct((E, D), jnp.float32),
        mesh=mesh,
        scratch_types=[
            pltpu.VMEM((NCH, CH), jnp.int32),
            pltpu.VMEM((GRP, CH, D), jnp.float32),
            pltpu.SemaphoreType.DMA,
        ],
    )
    def gk(table, idx, out, idx_v, buf, gsem):
        c = lax.axis_index("c")
        s = lax.axis_index("s")
        wid = s * NC + c
        base = wid * EPWG
        pltpu.sync_copy(idx.at[wid], idx_v)

        def group(gi, _):
            j0 = gi * GRP
            descs = [
                pltpu.async_copy(table.at[idx_v.at[j0 + b]], buf.at[b], gsem)
                for b in range(GRP)
            ]
            for b in range(GRP):
                descs[b].wait()
                pltpu.sync_copy(buf.at[b],
                                out.at[pl.ds(base + (j0 + b) * CH, CH)])
            return 0

        lax.fori_loop(0, NGRP, group, 0)

    return gk(xa, idx3)


def _sc_deg(idx3, zeros_d, ones_ch):
    mesh = plsc.VectorSubcoreMesh(core_axis_name="c", subcore_axis_name="s",
                                  num_cores=NC, num_subcores=NS)

    @functools.partial(
        pl.kernel,
        out_type=jax.ShapeDtypeStruct((NC, N, DW), jnp.float32),
        mesh=mesh,
        scratch_types=[
            pltpu.VMEM((NCHD, CHD), jnp.int32),
            pltpu.VMEM((CHD, DW), jnp.float32),
            pltpu.VMEM_SHARED((N, DW), jnp.float32),
        ],
    )
    def dk(idx, zd, ones_h, degp, idx_v, ones_v, shared_d):
        c = lax.axis_index("c")
        s = lax.axis_index("s")
        wid = s * NC + c

        @pl.when(s < NRW)
        def _():
            pltpu.sync_copy(zd.at[pl.ds(s * RPW, RPW)],
                            shared_d.at[pl.ds(s * RPW, RPW)])

        pltpu.sync_copy(idx.at[wid], idx_v)
        pltpu.sync_copy(ones_h, ones_v)
        plsc.subcore_barrier()

        def chunk(j, _):
            pltpu.sync_copy(ones_v, shared_d.at[idx_v.at[j]], add=True)
            return 0

        lax.fori_loop(0, NCHD, chunk, 0)
        plsc.subcore_barrier()

        @pl.when(s < NRW)
        def _():
            pltpu.sync_copy(shared_d.at[pl.ds(s * RPW, RPW)],
                            degp.at[c, pl.ds(s * RPW, RPW)])

    return dk(idx3, zeros_d, ones_ch)


def _edge_body(emb_ref, g_ref, w_ref, b_ref, o_ref):
    v = jnp.dot(emb_ref[...], w_ref[...],
                preferred_element_type=jnp.float32)
    o_ref[...] = _gelu(v + b_ref[...] + g_ref[...])


def _tc_edge(emb, g, w1b, b1, sl):
    be = 1280
    off = sl * (E2 // be)
    return pl.pallas_call(
        _edge_body,
        grid=(E2 // be,),
        in_specs=[pl.BlockSpec((be, D), lambda i: (i + off, 0)),
                  pl.BlockSpec((be, D), lambda i: (i + off, 0)),
                  pl.BlockSpec((D, D), lambda i: (0, 0)),
                  pl.BlockSpec((1, D), lambda i: (0, 0))],
        out_specs=pl.BlockSpec((be, D), lambda i: (i, 0)),
        out_shape=jax.ShapeDtypeStruct((E2, D), jnp.float32),
    )(emb, g, w1b, b1)


def _sc_scatter(t, idx3, zeros_g):
    mesh = plsc.VectorSubcoreMesh(core_axis_name="c", subcore_axis_name="s",
                                  num_cores=NC, num_subcores=NS)

    @functools.partial(
        pl.kernel,
        out_type=jax.ShapeDtypeStruct((NC, N, D), jnp.float32),
        mesh=mesh,
        scratch_types=[
            pltpu.VMEM((NCHS, CHS), jnp.int32),
            pltpu.VMEM((CHS, D), jnp.float32),
            pltpu.VMEM((CHS, D), jnp.float32),
            pltpu.VMEM_SHARED((N, D), jnp.float32),
            pltpu.SemaphoreType.DMA,
            pltpu.SemaphoreType.DMA,
        ],
    )
    def sk(t_hbm, idx, zg, gp, idx_v, buf0, buf1, shared_g, sem0, sem1):
        c = lax.axis_index("c")
        s = lax.axis_index("s")
        wid = s * NC + c
        base = wid * EPW

        @pl.when(s < NRW)
        def _():
            pltpu.sync_copy(zg.at[pl.ds(s * RPW, RPW)],
                            shared_g.at[pl.ds(s * RPW, RPW)])

        pltpu.sync_copy(idx.at[wid], idx_v)
        plsc.subcore_barrier()

        pltpu.async_copy(t_hbm.at[pl.ds(base, CHS)], buf0, sem0)
        pltpu.async_copy(t_hbm.at[pl.ds(base + CHS, CHS)], buf1, sem1)

        def pair(hi, _):
            j0 = 2 * hi
            pltpu.make_async_copy(t_hbm.at[pl.ds(base + j0 * CHS, CHS)],
                                  buf0, sem0).wait()
            pltpu.sync_copy(buf0, shared_g.at[idx_v.at[j0]], add=True)

            @pl.when(j0 + 2 < NCHS)
            def _():
                pltpu.async_copy(t_hbm.at[pl.ds(base + (j0 + 2) * CHS, CHS)],
                                 buf0, sem0)

            pltpu.make_async_copy(t_hbm.at[pl.ds(base + (j0 + 1) * CHS, CHS)],
                                  buf1, sem1).wait()
            pltpu.sync_copy(buf1, shared_g.at[idx_v.at[j0 + 1]], add=True)

            @pl.when(j0 + 3 < NCHS)
            def _():
                pltpu.async_copy(t_hbm.at[pl.ds(base + (j0 + 3) * CHS, CHS)],
                                 buf1, sem1)

            return 0

        lax.fori_loop(0, NPAIR, pair, 0)
        pltpu.make_async_copy(t_hbm.at[pl.ds(base + (NCHS - 1) * CHS, CHS)],
                              buf0, sem0).wait()
        pltpu.sync_copy(buf0, shared_g.at[idx_v.at[NCHS - 1]], add=True)
        plsc.subcore_barrier()

        @pl.when(s < NRW)
        def _():
            pltpu.sync_copy(shared_g.at[pl.ds(s * RPW, RPW)],
                            gp.at[c, pl.ds(s * RPW, RPW)])

    return sk(t, idx3, zeros_g)


def _node_body(x_ref, ga0, ga1, gb0, gb1, d0, d1, w2, b2, w3a, w3b, b3,
               w4, b4, gam, bet, o_ref):
    big_g = (ga0[...] + ga1[...]) + (gb0[...] + gb1[...])
    d = d0[:, 0:1] + d1[:, 0:1]
    agg = jnp.dot(big_g, w2[...], preferred_element_type=jnp.float32)
    agg = agg / jnp.maximum(d, 1.0) + jnp.where(d > 0.0, 1.0, 0.0) * b2[...]
    u = (jnp.dot(x_ref[...], w3a[...], preferred_element_type=jnp.float32)
         + jnp.dot(agg, w3b[...], preferred_element_type=jnp.float32)
         + b3[...])
    u = _gelu(u)
    h = jnp.dot(u, w4[...], preferred_element_type=jnp.float32) + b4[...]
    y = x_ref[...] + h
    mu = jnp.mean(y, axis=-1, keepdims=True)
    yc = y - mu
    var = jnp.mean(yc * yc, axis=-1, keepdims=True)
    o_ref[...] = yc * lax.rsqrt(var + 1e-5) * gam[...] + bet[...]


def _tc_node(x, ga0, ga1, gb0, gb1, d0, d1, w2, b2, w3a, w3b, b3, w4, b4,
             gam, bet):
    bn = 2000
    full = lambda r, c: pl.BlockSpec((r, c), lambda i: (0, 0))
    row = lambda cols: pl.BlockSpec((bn, cols), lambda i: (i, 0))
    return pl.pallas_call(
        _node_body,
        grid=(N // bn,),
        in_specs=[row(D), row(D), row(D), row(D), row(D), row(DW), row(DW),
                  full(D, D), full(1, D), full(D, DFF), full(D, DFF),
                  full(1, DFF), full(DFF, D), full(1, D),
                  full(1, D), full(1, D)],
        out_specs=pl.BlockSpec((bn, D), lambda i: (i, 0)),
        out_shape=jax.ShapeDtypeStruct((N, D), jnp.float32),
    )(x, ga0, ga1, gb0, gb1, d0, d1, w2, b2, w3a, w3b, b3, w4, b4, gam, bet)


def kernel(x, edge_src, edge_dst, edge_emb, W1, b1, W2, b2, W3, b3, W4, b4,
           gamma, beta):
    w1a, w1b = W1[:D], W1[D:]
    w3a, w3b = W3[:D], W3[D:]
    src3 = edge_src.reshape(NW, NCH, CH)
    dst4 = edge_dst.reshape(NSL, NW, NCHS, CHS)
    dst3d = edge_dst.reshape(NW, NCHD, CHD)
    zeros_g = jnp.zeros((N, D), jnp.float32)
    zeros_d = jnp.zeros((N, DW), jnp.float32)
    ones_ch = jnp.ones((CHD, DW), jnp.float32)
    b1r = b1.reshape(1, D)

    xa = _tc_xa(x, w1a)
    g = _sc_gather(xa, src3)
    degp = _sc_deg(dst3d, zeros_d, ones_ch)
    ta = _tc_edge(edge_emb, g, w1b, b1r, 0)
    gpa = _sc_scatter(ta, dst4[0], zeros_g)
    tb = _tc_edge(edge_emb, g, w1b, b1r, 1)
    gpb = _sc_scatter(tb, dst4[1], zeros_g)
    return _tc_node(x, gpa[0], gpa[1], gpb[0], gpb[1], degp[0], degp[1],
                    W2, b2.reshape(1, D), w3a, w3b, b3.reshape(1, DFF),
                    W4, b4.reshape(1, D), gamma.reshape(1, D),
                    beta.reshape(1, D))


# speedup vs baseline: 1.0626x; 1.0626x over previous
import functools

import jax
import jax.numpy as jnp
from jax import lax
from jax.experimental import pallas as pl
from jax.experimental.pallas import tpu as pltpu
from jax.experimental.pallas import tpu_sc as plsc

N, E, D, DFF = 10000, 320000, 128, 512
NC, NS = 2, 16
NW = NC * NS
NSL = 2
E2 = E // NSL
EPW = E2 // NW
CH = 40
NCH = EPW // CH
GRP = 5
NGRP = NCH // GRP
CHS = 40
NCHS = EPW // CHS
NPAIR = (NCHS - 1) // 2
EPWD = E // NW
CHD = 80
NCHD = EPWD // CHD
DW = 128
RPW = 1000
NRW = N // RPW

_SQRT_HALF = 0.7071067811865476


def _gelu(v):
    return 0.5 * v * (1.0 + lax.erf(v * _SQRT_HALF))


def _xa_body(x_ref, w_ref, o_ref):
    o_ref[...] = jnp.dot(x_ref[...], w_ref[...],
                         preferred_element_type=jnp.float32)


def _tc_xa(x, w1a):
    bn = 2000
    return pl.pallas_call(
        _xa_body,
        grid=(N // bn,),
        in_specs=[pl.BlockSpec((bn, D), lambda i: (i, 0)),
                  pl.BlockSpec((D, D), lambda i: (0, 0))],
        out_specs=pl.BlockSpec((bn, D), lambda i: (i, 0)),
        out_shape=jax.ShapeDtypeStruct((N, D), jnp.float32),
    )(x, w1a)


def _sc_gather(xa, idx3):
    mesh = plsc.VectorSubcoreMesh(core_axis_name="c", subcore_axis_name="s",
                                  num_cores=NC, num_subcores=NS)

    @functools.partial(
        pl.kernel,
        out_type=jax.ShapeDtypeStruct((E2, D), jnp.float32),
        mesh=mesh,
        scratch_types=[
            pltpu.VMEM((NCH, CH), jnp.int32),
            pltpu.VMEM((GRP, CH, D), jnp.float32),
            pltpu.SemaphoreType.DMA,
        ],
    )
    def gk(table, idx, out, idx_v, buf, gsem):
        c = lax.axis_index("c")
        s = lax.axis_index("s")
        wid = s * NC + c
        base = wid * EPW
        pltpu.sync_copy(idx.at[wid], idx_v)

        def group(gi, _):
            j0 = gi * GRP
            descs = [
                pltpu.async_copy(table.at[idx_v.at[j0 + b]], buf.at[b], gsem)
                for b in range(GRP)
            ]
            for b in range(GRP):
                descs[b].wait()
                pltpu.sync_copy(buf.at[b],
                                out.at[pl.ds(base + (j0 + b) * CH, CH)])
            return 0

        lax.fori_loop(0, NGRP, group, 0)

    return gk(xa, idx3)


def _sc_deg(idx3, zeros_d, ones_ch):
    mesh = plsc.VectorSubcoreMesh(core_axis_name="c", subcore_axis_name="s",
                                  num_cores=NC, num_subcores=NS)

    @functools.partial(
        pl.kernel,
        out_type=jax.ShapeDtypeStruct((NC, N, DW), jnp.float32),
        mesh=mesh,
        scratch_types=[
            pltpu.VMEM((NCHD, CHD), jnp.int32),
            pltpu.VMEM((CHD, DW), jnp.float32),
            pltpu.VMEM_SHARED((N, DW), jnp.float32),
        ],
    )
    def dk(idx, zd, ones_h, degp, idx_v, ones_v, shared_d):
        c = lax.axis_index("c")
        s = lax.axis_index("s")
        wid = s * NC + c

        @pl.when(s < NRW)
        def _():
            pltpu.sync_copy(zd.at[pl.ds(s * RPW, RPW)],
                            shared_d.at[pl.ds(s * RPW, RPW)])

        pltpu.sync_copy(idx.at[wid], idx_v)
        pltpu.sync_copy(ones_h, ones_v)
        plsc.subcore_barrier()

        def chunk(j, _):
            pltpu.sync_copy(ones_v, shared_d.at[idx_v.at[j]], add=True)
            return 0

        lax.fori_loop(0, NCHD, chunk, 0)
        plsc.subcore_barrier()

        @pl.when(s < NRW)
        def _():
            pltpu.sync_copy(shared_d.at[pl.ds(s * RPW, RPW)],
                            degp.at[c, pl.ds(s * RPW, RPW)])

    return dk(idx3, zeros_d, ones_ch)


def _edge_body(emb_ref, g_ref, w_ref, b_ref, o_ref):
    v = jnp.dot(emb_ref[...], w_ref[...],
                preferred_element_type=jnp.float32)
    o_ref[...] = _gelu(v + b_ref[...] + g_ref[...])


def _edge_body_dep(emb_ref, g_ref, w_ref, b_ref, d_ref, o_ref):
    del d_ref
    v = jnp.dot(emb_ref[...], w_ref[...],
                preferred_element_type=jnp.float32)
    o_ref[...] = _gelu(v + b_ref[...] + g_ref[...])


def _tc_edge(emb, g, w1b, b1, sl, dep=None):
    be = 4000
    off = sl * (E2 // be)
    specs = [pl.BlockSpec((be, D), lambda i: (i + off, 0)),
             pl.BlockSpec((be, D), lambda i: (i, 0)),
             pl.BlockSpec((D, D), lambda i: (0, 0)),
             pl.BlockSpec((1, D), lambda i: (0, 0))]
    args = [emb, g, w1b, b1]
    body = _edge_body
    if dep is not None:
        specs.append(pl.BlockSpec((1, 8, DW), lambda i: (0, 0, 0)))
        args.append(dep)
        body = _edge_body_dep
    return pl.pallas_call(
        body,
        grid=(E2 // be,),
        in_specs=specs,
        out_specs=pl.BlockSpec((be, D), lambda i: (i, 0)),
        out_shape=jax.ShapeDtypeStruct((E2, D), jnp.float32),
    )(*args)


def _sc_scatter(t, idx3, zeros_g):
    mesh = plsc.VectorSubcoreMesh(core_axis_name="c", subcore_axis_name="s",
                                  num_cores=NC, num_subcores=NS)

    @functools.partial(
        pl.kernel,
        out_type=jax.ShapeDtypeStruct((NC, N, D), jnp.float32),
        mesh=mesh,
        scratch_types=[
            pltpu.VMEM((NCHS, CHS), jnp.int32),
            pltpu.VMEM((CHS, D), jnp.float32),
            pltpu.VMEM((CHS, D), jnp.float32),
            pltpu.VMEM_SHARED((N, D), jnp.float32),
            pltpu.SemaphoreType.DMA,
            pltpu.SemaphoreType.DMA,
        ],
    )
    def sk(t_hbm, idx, zg, gp, idx_v, buf0, buf1, shared_g, sem0, sem1):
        c = lax.axis_index("c")
        s = lax.axis_index("s")
        wid = s * NC + c
        base = wid * EPW

        @pl.when(s < NRW)
        def _():
            pltpu.sync_copy(zg.at[pl.ds(s * RPW, RPW)],
                            shared_g.at[pl.ds(s * RPW, RPW)])

        pltpu.sync_copy(idx.at[wid], idx_v)
        plsc.subcore_barrier()

        pltpu.async_copy(t_hbm.at[pl.ds(base, CHS)], buf0, sem0)
        pltpu.async_copy(t_hbm.at[pl.ds(base + CHS, CHS)], buf1, sem1)

        def pair(hi, _):
            j0 = 2 * hi
            pltpu.make_async_copy(t_hbm.at[pl.ds(base + j0 * CHS, CHS)],
                                  buf0, sem0).wait()
            pltpu.sync_copy(buf0, shared_g.at[idx_v.at[j0]], add=True)

            @pl.when(j0 + 2 < NCHS)
            def _():
                pltpu.async_copy(t_hbm.at[pl.ds(base + (j0 + 2) * CHS, CHS)],
                                 buf0, sem0)

            pltpu.make_async_copy(t_hbm.at[pl.ds(base + (j0 + 1) * CHS, CHS)],
                                  buf1, sem1).wait()
            pltpu.sync_copy(buf1, shared_g.at[idx_v.at[j0 + 1]], add=True)

            @pl.when(j0 + 3 < NCHS)
            def _():
                pltpu.async_copy(t_hbm.at[pl.ds(base + (j0 + 3) * CHS, CHS)],
                                 buf1, sem1)

            return 0

        lax.fori_loop(0, NPAIR, pair, 0)
        pltpu.make_async_copy(t_hbm.at[pl.ds(base + (NCHS - 1) * CHS, CHS)],
                              buf0, sem0).wait()
        pltpu.sync_copy(buf0, shared_g.at[idx_v.at[NCHS - 1]], add=True)
        plsc.subcore_barrier()

        @pl.when(s < NRW)
        def _():
            pltpu.sync_copy(shared_g.at[pl.ds(s * RPW, RPW)],
                            gp.at[c, pl.ds(s * RPW, RPW)])

    return sk(t, idx3, zeros_g)


def _node_body(x_ref, ga0, ga1, gb0, gb1, d0, d1, w2, b2, w3a, w3b, b3,
               w4, b4, gam, bet, o_ref):
    big_g = (ga0[...] + ga1[...]) + (gb0[...] + gb1[...])
    d = d0[:, 0:1] + d1[:, 0:1]
    agg = jnp.dot(big_g, w2[...], preferred_element_type=jnp.float32)
    agg = agg / jnp.maximum(d, 1.0) + jnp.where(d > 0.0, 1.0, 0.0) * b2[...]
    u = (jnp.dot(x_ref[...], w3a[...], preferred_element_type=jnp.float32)
         + jnp.dot(agg, w3b[...], preferred_element_type=jnp.float32)
         + b3[...])
    u = _gelu(u)
    h = jnp.dot(u, w4[...], preferred_element_type=jnp.float32) + b4[...]
    y = x_ref[...] + h
    mu = jnp.mean(y, axis=-1, keepdims=True)
    yc = y - mu
    var = jnp.mean(yc * yc, axis=-1, keepdims=True)
    o_ref[...] = yc * lax.rsqrt(var + 1e-5) * gam[...] + bet[...]


def _tc_node(x, ga0, ga1, gb0, gb1, d0, d1, w2, b2, w3a, w3b, b3, w4, b4,
             gam, bet):
    bn = 2000
    full = lambda r, c: pl.BlockSpec((r, c), lambda i: (0, 0))
    row = lambda cols: pl.BlockSpec((bn, cols), lambda i: (i, 0))
    return pl.pallas_call(
        _node_body,
        grid=(N // bn,),
        in_specs=[row(D), row(D), row(D), row(D), row(D), row(DW), row(DW),
                  full(D, D), full(1, D), full(D, DFF), full(D, DFF),
                  full(1, DFF), full(DFF, D), full(1, D),
                  full(1, D), full(1, D)],
        out_specs=pl.BlockSpec((bn, D), lambda i: (i, 0)),
        out_shape=jax.ShapeDtypeStruct((N, D), jnp.float32),
    )(x, ga0, ga1, gb0, gb1, d0, d1, w2, b2, w3a, w3b, b3, w4, b4, gam, bet)


def kernel(x, edge_src, edge_dst, edge_emb, W1, b1, W2, b2, W3, b3, W4, b4,
           gamma, beta):
    w1a, w1b = W1[:D], W1[D:]
    w3a, w3b = W3[:D], W3[D:]
    src4 = edge_src.reshape(NSL, NW, NCH, CH)
    dst4 = edge_dst.reshape(NSL, NW, NCHS, CHS)
    dst3d = edge_dst.reshape(NW, NCHD, CHD)
    zeros_g = jnp.zeros((N, D), jnp.float32)
    zeros_d = jnp.zeros((N, DW), jnp.float32)
    ones_ch = jnp.ones((CHD, DW), jnp.float32)
    b1r = b1.reshape(1, D)

    xa = _tc_xa(x, w1a)
    ga = _sc_gather(xa, src4[0])
    gb = _sc_gather(xa, src4[1])
    degp = _sc_deg(dst3d, zeros_d, ones_ch)
    ta = _tc_edge(edge_emb, ga, w1b, b1r, 0)
    gpa = _sc_scatter(ta, dst4[0], zeros_g)
    tb = _tc_edge(edge_emb, gb, w1b, b1r, 1, dep=degp)
    gpb = _sc_scatter(tb, dst4[1], zeros_g)
    return _tc_node(x, gpa[0], gpa[1], gpb[0], gpb[1], degp[0], degp[1],
                    W2, b2.reshape(1, D), w3a, w3b, b3.reshape(1, DFF),
                    W4, b4.reshape(1, D), gamma.reshape(1, D),
                    beta.reshape(1, D))
